# baseline (speedup 1.0000x reference)
import functools
import math

import jax
import jax.numpy as jnp
from jax import lax
from jax.experimental import pallas as pl
from jax.experimental.pallas import tpu as pltpu
from jax.experimental.pallas import tpu_sc as plsc

_N = 2048
_V = 32000
_PAD = 0
_SMOOTH = 0.1
_CONF = 1.0 - _SMOOTH
_S = _SMOOTH / (_V - 2)
_K = (_V - 2) * _S * math.log(_S) + _CONF * math.log(_CONF)

_R = 256
_C = 6400

_NW = 32
_NSC = 256
_NTC = _N - _NSC
_RPW = _NSC // _NW
_L = 16


def _loss_body(t_ref, x_ref, o_ref):
    i = pl.program_id(0)
    j = pl.program_id(1)

    @pl.when((i == 0) & (j == 0))
    def _init():
        o_ref[0, 0] = 0.0

    t = t_ref[...]
    valid = (t != _PAD)
    xb = x_ref[...]

    tloc = jnp.where(valid, t - j * _C, -1)
    iota = jax.lax.broadcasted_iota(jnp.int32, (_R, _C), 1)
    gath = jnp.sum(jnp.where(iota == tloc, xb, 0.0))

    rows = jnp.sum(xb, axis=1, keepdims=True)
    rsum = jnp.sum(jnp.where(valid, rows, 0.0))

    partial = jnp.float32(_S - _CONF) * gath - jnp.float32(_S) * rsum

    @pl.when(j == 0)
    def _const():
        x0 = xb[:, 0:1]
        o_ref[0, 0] += jnp.sum(
            jnp.where(valid, jnp.float32(_K) + jnp.float32(_S) * x0, 0.0))

    o_ref[0, 0] += partial


def _tc_loss(x, t2):
    nr = _NTC // _R
    nc = _V // _C
    out = pl.pallas_call(
        _loss_body,
        grid=(nr, nc),
        in_specs=[
            pl.BlockSpec((_R, 1), lambda i, j: (i, 0)),
            pl.BlockSpec((_R, _C), lambda i, j: (i, j)),
        ],
        out_specs=pl.BlockSpec((1, 1), lambda i, j: (0, 0),
                               memory_space=pltpu.SMEM),
        out_shape=jax.ShapeDtypeStruct((1, 1), jnp.float32),
        compiler_params=pltpu.CompilerParams(
            dimension_semantics=("arbitrary", "arbitrary")),
    )(t2, x)
    return out[0, 0]


def _sc_rows_body(x_hbm, t_hbm, out_hbm, t_v, buf0, buf1, acc_v, g_v,
                  sem0, sem1, semg):
    wid = lax.axis_index("s") * 2 + lax.axis_index("c")
    base = _NTC + wid * _RPW

    pltpu.sync_copy(t_hbm.at[pl.ds(base, _L)], t_v)
    lanes = lax.iota(jnp.int32, _L)
    onehot0 = lanes == 0

    bufs = (buf0, buf1)
    sems = (sem0, sem1)
    descs = [None, None]
    descs[0] = pltpu.async_copy(x_hbm.at[base, :], buf0, sem0)

    vtotal = jnp.zeros((_L,), jnp.float32)
    for r in range(_RPW):
        cur = bufs[r % 2]
        if r + 1 < _RPW:
            descs[(r + 1) % 2] = pltpu.async_copy(
                x_hbm.at[base + r + 1, :], bufs[(r + 1) % 2], sems[(r + 1) % 2])
        descs[r % 2].wait()

        tb = lax.gather(
            t_v[...], jnp.full((_L, 1), r, jnp.int32),
            dimension_numbers=lax.GatherDimensionNumbers(
                offset_dims=(), collapsed_slice_dims=(0,),
                start_index_map=(0,)),
            slice_sizes=(1,),
            mode=lax.GatherScatterMode.PROMISE_IN_BOUNDS)
        validf = jnp.sign(tb.astype(jnp.float32))

        ktv = (tb >> 4).astype(jnp.float32)

        def _red(k, carry):
            vac, gacc, kcnt = carry
            chunk = cur[pl.ds(k * _L, _L)]
            hit = jnp.float32(1.0) - jnp.sign(jnp.abs(kcnt - ktv))
            return (vac + chunk, gacc + hit * chunk, kcnt + jnp.float32(1.0))
        vac, gacc, _ = lax.fori_loop(
            0, _V // _L, _red,
            (jnp.zeros((_L,), jnp.float32), jnp.zeros((_L,), jnp.float32),
             jnp.zeros((_L,), jnp.float32)),
            unroll=8)

        x0v = cur[pl.ds(0, _L)]

        onehot_t = jnp.float32(1.0) - jnp.sign(
            jnp.abs(lanes - (tb & jnp.int32(15))).astype(jnp.float32))

        contrib = (jnp.where(onehot0,
                             jnp.float32(_K) + jnp.float32(_S) * x0v,
                             jnp.float32(0.0))
                   + jnp.float32(_S - _CONF) * onehot_t * gacc
                   - jnp.float32(_S) * vac)
        vtotal = vtotal + validf * contrib

    acc_v[...] = vtotal
    pltpu.sync_copy(acc_v, out_hbm.at[wid])


def _sc_loss(x, t32):
    mesh = plsc.VectorSubcoreMesh(core_axis_name="c", subcore_axis_name="s")
    fn = functools.partial(
        pl.kernel,
        mesh=mesh,
        out_type=jax.ShapeDtypeStruct((_NW, _L), jnp.float32),
        scratch_types=[
            pltpu.VMEM((_L,), jnp.int32),
            pltpu.VMEM((_V,), jnp.float32),
            pltpu.VMEM((_V,), jnp.float32),
            pltpu.VMEM((_L,), jnp.float32),
            pltpu.VMEM((_L,), jnp.float32),
            pltpu.SemaphoreType.DMA,
            pltpu.SemaphoreType.DMA,
            pltpu.SemaphoreType.DMA,
        ],
    )(_sc_rows_body)
    return fn(x, t32)


def kernel(x, target):
    t32 = target.astype(jnp.int32)
    t2 = t32[:_NTC].reshape(_NTC, 1)
    sc_part = _sc_loss(x, t32)
    tc_part = _tc_loss(x, t2)
    return tc_part + jnp.sum(sc_part)


# speedup vs baseline: 1.0646x; 1.0646x over previous
import functools
import math

import jax
import jax.numpy as jnp
from jax import lax
from jax.experimental import pallas as pl
from jax.experimental.pallas import tpu as pltpu
from jax.experimental.pallas import tpu_sc as plsc

_N = 2048
_V = 32000
_PAD = 0
_SMOOTH = 0.1
_CONF = 1.0 - _SMOOTH
_S = _SMOOTH / (_V - 2)
_K = (_V - 2) * _S * math.log(_S) + _CONF * math.log(_CONF)

_R = 256
_C = 6400

_NW = 32
_NSC = 512
_NTC = _N - _NSC
_RPW = _NSC // _NW
_L = 16


def _loss_body(t_ref, x_ref, o_ref):
    i = pl.program_id(0)
    j = pl.program_id(1)

    @pl.when((i == 0) & (j == 0))
    def _init():
        o_ref[0, 0] = 0.0

    t = t_ref[...]
    valid = (t != _PAD)
    xb = x_ref[...]

    tloc = jnp.where(valid, t - j * _C, -1)
    iota = jax.lax.broadcasted_iota(jnp.int32, (_R, _C), 1)
    gath = jnp.sum(jnp.where(iota == tloc, xb, 0.0))

    rows = jnp.sum(xb, axis=1, keepdims=True)
    rsum = jnp.sum(jnp.where(valid, rows, 0.0))

    partial = jnp.float32(_S - _CONF) * gath - jnp.float32(_S) * rsum

    @pl.when(j == 0)
    def _const():
        x0 = xb[:, 0:1]
        o_ref[0, 0] += jnp.sum(
            jnp.where(valid, jnp.float32(_K) + jnp.float32(_S) * x0, 0.0))

    o_ref[0, 0] += partial


def _tc_loss(x, t2):
    nr = _NTC // _R
    nc = _V // _C
    out = pl.pallas_call(
        _loss_body,
        grid=(nr, nc),
        in_specs=[
            pl.BlockSpec((_R, 1), lambda i, j: (i, 0)),
            pl.BlockSpec((_R, _C), lambda i, j: (i, j)),
        ],
        out_specs=pl.BlockSpec((1, 1), lambda i, j: (0, 0),
                               memory_space=pltpu.SMEM),
        out_shape=jax.ShapeDtypeStruct((1, 1), jnp.float32),
        compiler_params=pltpu.CompilerParams(
            dimension_semantics=("arbitrary", "arbitrary")),
    )(t2, x)
    return out[0, 0]


def _sc_rows_body(x_hbm, t_hbm, out_hbm, t_v, buf0, buf1, acc_v, g_v,
                  sem0, sem1, semg):
    wid = lax.axis_index("s") * 2 + lax.axis_index("c")
    base = _NTC + wid * _RPW

    pltpu.sync_copy(t_hbm.at[pl.ds(base, _L)], t_v)
    lanes = lax.iota(jnp.int32, _L)
    onehot0 = lanes == 0

    bufs = (buf0, buf1)
    sems = (sem0, sem1)
    descs = [None, None]
    descs[0] = pltpu.async_copy(x_hbm.at[base, :], buf0, sem0)

    vtotal = jnp.zeros((_L,), jnp.float32)
    for r in range(_RPW):
        cur = bufs[r % 2]
        if r + 1 < _RPW:
            descs[(r + 1) % 2] = pltpu.async_copy(
                x_hbm.at[base + r + 1, :], bufs[(r + 1) % 2], sems[(r + 1) % 2])
        descs[r % 2].wait()

        tb = lax.gather(
            t_v[...], jnp.full((_L, 1), r, jnp.int32),
            dimension_numbers=lax.GatherDimensionNumbers(
                offset_dims=(), collapsed_slice_dims=(0,),
                start_index_map=(0,)),
            slice_sizes=(1,),
            mode=lax.GatherScatterMode.PROMISE_IN_BOUNDS)
        validf = jnp.sign(tb.astype(jnp.float32))

        ktv = (tb >> 4).astype(jnp.float32)

        def _red(k, carry):
            vac, gacc, kcnt = carry
            chunk = cur[pl.ds(k * _L, _L)]
            hit = jnp.float32(1.0) - jnp.sign(jnp.abs(kcnt - ktv))
            return (vac + chunk, gacc + hit * chunk, kcnt + jnp.float32(1.0))
        vac, gacc, _ = lax.fori_loop(
            0, _V // _L, _red,
            (jnp.zeros((_L,), jnp.float32), jnp.zeros((_L,), jnp.float32),
             jnp.zeros((_L,), jnp.float32)),
            unroll=8)

        x0v = cur[pl.ds(0, _L)]

        onehot_t = jnp.float32(1.0) - jnp.sign(
            jnp.abs(lanes - (tb & jnp.int32(15))).astype(jnp.float32))

        contrib = (jnp.where(onehot0,
                             jnp.float32(_K) + jnp.float32(_S) * x0v,
                             jnp.float32(0.0))
                   + jnp.float32(_S - _CONF) * onehot_t * gacc
                   - jnp.float32(_S) * vac)
        vtotal = vtotal + validf * contrib

    acc_v[...] = vtotal
    pltpu.sync_copy(acc_v, out_hbm.at[wid])


def _sc_loss(x, t32):
    mesh = plsc.VectorSubcoreMesh(core_axis_name="c", subcore_axis_name="s")
    fn = functools.partial(
        pl.kernel,
        mesh=mesh,
        out_type=jax.ShapeDtypeStruct((_NW, _L), jnp.float32),
        scratch_types=[
            pltpu.VMEM((_L,), jnp.int32),
            pltpu.VMEM((_V,), jnp.float32),
            pltpu.VMEM((_V,), jnp.float32),
            pltpu.VMEM((_L,), jnp.float32),
            pltpu.VMEM((_L,), jnp.float32),
            pltpu.SemaphoreType.DMA,
            pltpu.SemaphoreType.DMA,
            pltpu.SemaphoreType.DMA,
        ],
    )(_sc_rows_body)
    return fn(x, t32)


def kernel(x, target):
    t32 = target.astype(jnp.int32)
    t2 = t32[:_NTC].reshape(_NTC, 1)
    sc_part = _sc_loss(x, t32)
    tc_part = _tc_loss(x, t2)
    return tc_part + jnp.sum(sc_part)


# speedup vs baseline: 1.0673x; 1.0026x over previous
import functools
import math

import jax
import jax.numpy as jnp
from jax import lax
from jax.experimental import pallas as pl
from jax.experimental.pallas import tpu as pltpu
from jax.experimental.pallas import tpu_sc as plsc

_N = 2048
_V = 32000
_PAD = 0
_SMOOTH = 0.1
_CONF = 1.0 - _SMOOTH
_S = _SMOOTH / (_V - 2)
_K = (_V - 2) * _S * math.log(_S) + _CONF * math.log(_CONF)

_R = 256
_C = 6400

_NW = 32
_NSC = 512
_NTC = _N - _NSC
_RPW = _NSC // _NW
_L = 16


def _loss_body(t_ref, x_ref, o_ref):
    i = pl.program_id(0)
    j = pl.program_id(1)

    @pl.when((i == 0) & (j == 0))
    def _init():
        o_ref[0, 0] = 0.0

    t = t_ref[...]
    valid = (t != _PAD)
    xb = x_ref[...]

    tloc = jnp.where(valid, t - j * _C, -1)
    iota = jax.lax.broadcasted_iota(jnp.int32, (_R, _C), 1)
    gath = jnp.sum(jnp.where(iota == tloc, xb, 0.0))

    rows = jnp.sum(xb, axis=1, keepdims=True)
    rsum = jnp.sum(jnp.where(valid, rows, 0.0))

    partial = jnp.float32(_S - _CONF) * gath - jnp.float32(_S) * rsum

    @pl.when(j == 0)
    def _const():
        x0 = xb[:, 0:1]
        o_ref[0, 0] += jnp.sum(
            jnp.where(valid, jnp.float32(_K) + jnp.float32(_S) * x0, 0.0))

    o_ref[0, 0] += partial


def _tc_loss(x, t2):
    nr = _NTC // _R
    nc = _V // _C
    out = pl.pallas_call(
        _loss_body,
        grid=(nr, nc),
        in_specs=[
            pl.BlockSpec((_R, 1), lambda i, j: (i, 0)),
            pl.BlockSpec((_R, _C), lambda i, j: (i, j)),
        ],
        out_specs=pl.BlockSpec((1, 1), lambda i, j: (0, 0),
                               memory_space=pltpu.SMEM),
        out_shape=jax.ShapeDtypeStruct((1, 1), jnp.float32),
        compiler_params=pltpu.CompilerParams(
            dimension_semantics=("arbitrary", "arbitrary")),
    )(t2, x)
    return out[0, 0]


def _sc_rows_body(x_hbm, t_hbm, out_hbm, t_v, buf0, buf1, acc_v, g_v,
                  sem0, sem1, semg):
    wid = lax.axis_index("s") * 2 + lax.axis_index("c")
    base = _NTC + wid * _RPW

    pltpu.sync_copy(t_hbm.at[pl.ds(base, _L)], t_v)
    lanes = lax.iota(jnp.int32, _L)
    onehot0 = lanes == 0

    bufs = (buf0, buf1)
    sems = (sem0, sem1)
    descs = [None, None]
    descs[0] = pltpu.async_copy(x_hbm.at[base, :], buf0, sem0)

    vtotal = jnp.zeros((_L,), jnp.float32)
    for r in range(_RPW):
        cur = bufs[r % 2]
        if r + 1 < _RPW:
            descs[(r + 1) % 2] = pltpu.async_copy(
                x_hbm.at[base + r + 1, :], bufs[(r + 1) % 2], sems[(r + 1) % 2])
        descs[r % 2].wait()

        tb = lax.gather(
            t_v[...], jnp.full((_L, 1), r, jnp.int32),
            dimension_numbers=lax.GatherDimensionNumbers(
                offset_dims=(), collapsed_slice_dims=(0,),
                start_index_map=(0,)),
            slice_sizes=(1,),
            mode=lax.GatherScatterMode.PROMISE_IN_BOUNDS)
        validf = jnp.sign(tb.astype(jnp.float32))

        ktv = (tb >> 4).astype(jnp.float32)

        def _red(k, carry):
            vac, gacc, kcnt = carry
            chunk = cur[pl.ds(k * _L, _L)]
            hit = jnp.float32(1.0) - jnp.sign(jnp.abs(kcnt - ktv))
            return (vac + chunk, gacc + hit * chunk, kcnt + jnp.float32(1.0))
        vac, gacc, _ = lax.fori_loop(
            0, _V // _L, _red,
            (jnp.zeros((_L,), jnp.float32), jnp.zeros((_L,), jnp.float32),
             jnp.zeros((_L,), jnp.float32)),
            unroll=16)

        x0v = cur[pl.ds(0, _L)]

        onehot_t = jnp.float32(1.0) - jnp.sign(
            jnp.abs(lanes - (tb & jnp.int32(15))).astype(jnp.float32))

        contrib = (jnp.where(onehot0,
                             jnp.float32(_K) + jnp.float32(_S) * x0v,
                             jnp.float32(0.0))
                   + jnp.float32(_S - _CONF) * onehot_t * gacc
                   - jnp.float32(_S) * vac)
        vtotal = vtotal + validf * contrib

    acc_v[...] = vtotal
    pltpu.sync_copy(acc_v, out_hbm.at[wid])


def _sc_loss(x, t32):
    mesh = plsc.VectorSubcoreMesh(core_axis_name="c", subcore_axis_name="s")
    fn = functools.partial(
        pl.kernel,
        mesh=mesh,
        out_type=jax.ShapeDtypeStruct((_NW, _L), jnp.float32),
        scratch_types=[
            pltpu.VMEM((_L,), jnp.int32),
            pltpu.VMEM((_V,), jnp.float32),
            pltpu.VMEM((_V,), jnp.float32),
            pltpu.VMEM((_L,), jnp.float32),
            pltpu.VMEM((_L,), jnp.float32),
            pltpu.SemaphoreType.DMA,
            pltpu.SemaphoreType.DMA,
            pltpu.SemaphoreType.DMA,
        ],
    )(_sc_rows_body)
    return fn(x, t32)


def kernel(x, target):
    t32 = target.astype(jnp.int32)
    t2 = t32[:_NTC].reshape(_NTC, 1)
    sc_part = _sc_loss(x, t32)
    tc_part = _tc_loss(x, t2)
    return tc_part + jnp.sum(sc_part)
